# baseline (speedup 1.0000x reference)
import functools

import jax
import jax.numpy as jnp
from jax import lax
from jax.experimental import pallas as pl
from jax.experimental.pallas import tpu as pltpu
from jax.experimental.pallas import tpu_sc as plsc

D = 64
TP = 128
V = 100000
B = 16384
NC = 2
NS = 16
NW = NC * NS
BPW = B // NW
CH = 128
NCH = BPW // CH
L = 16
NSTRIP = 782
KMAX = 25

_mesh = plsc.VectorSubcoreMesh(core_axis_name="c", subcore_axis_name="s")
_params = pltpu.CompilerParams(
    use_tc_tiling_on_sc=True,
    needs_layout_passes=False,
    disable_bounds_checks=True,
)


@functools.partial(
    pl.kernel,
    mesh=_mesh,
    out_type=jax.ShapeDtypeStruct((V, TP), jnp.float32),
    scratch_types=[
        pltpu.VMEM((2, D, CH), jnp.float32),
        pltpu.VMEM((2, CH, TP), jnp.float32),
        pltpu.SemaphoreType.DMA,
        pltpu.SemaphoreType.DMA,
        pltpu.SemaphoreType.DMA,
    ],
    compiler_params=_params,
)
def _relayout_kernel(tt_hbm, tpad_hbm, buf, tbuf, rsem, wsem, wsem2):
    wid = lax.axis_index("s") * NC + lax.axis_index("c")
    lane = lax.iota(jnp.int32, L)

    def strip_start(k):
        s = wid + 32 * k
        return 128 * s

    pltpu.async_copy(tt_hbm.at[:, pl.ds(strip_start(0), CH)], buf.at[0], rsem)
    for k in range(KMAX):
        s = wid + 32 * k
        cur = k % 2
        nxt = 1 - cur
        wsem_k = wsem if cur == 0 else wsem2

        if k >= 2:
            pltpu.make_async_copy(
                tbuf.at[cur], tpad_hbm.at[pl.ds(0, CH)],
                wsem_k,
            ).wait()

        @pl.when(s < NSTRIP)
        def _():
            @pl.when(s + 32 < NSTRIP)
            def _():
                pltpu.async_copy(
                    tt_hbm.at[:, pl.ds(strip_start(k + 1), CH)],
                    buf.at[nxt],
                    rsem,
                )

            pltpu.make_async_copy(
                tt_hbm.at[:, pl.ds(0, CH)], buf.at[cur], rsem
            ).wait()

            @plsc.parallel_loop(0, CH, unroll=4)
            def _(r):
                col = jnp.full((L,), r, jnp.int32)
                for g in range(D // L):
                    v = plsc.load_gather(buf.at[cur], [lane + g * L, col])
                    tbuf[cur, r, pl.ds(g * L, L)] = v

            @pl.when(s == NSTRIP - 1)
            def _():
                pltpu.async_copy(
                    tbuf.at[cur, pl.ds(0, 32)],
                    tpad_hbm.at[pl.ds(128 * (NSTRIP - 1), 32)],
                    wsem_k,
                )

            @pl.when(s < NSTRIP - 1)
            def _():
                pltpu.async_copy(
                    tbuf.at[cur],
                    tpad_hbm.at[pl.ds(strip_start(k), CH)],
                    wsem_k,
                )

    pltpu.make_async_copy(
        tbuf.at[(KMAX - 2) % 2], tpad_hbm.at[pl.ds(0, CH)],
        wsem if (KMAX - 2) % 2 == 0 else wsem2,
    ).wait()
    last_cur = (KMAX - 1) % 2
    last_sem = wsem if last_cur == 0 else wsem2
    s_last = wid + 32 * (KMAX - 1)

    @pl.when(s_last < NSTRIP - 1)
    def _():
        pltpu.make_async_copy(
            tbuf.at[last_cur], tpad_hbm.at[pl.ds(0, CH)],
            last_sem,
        ).wait()

    @pl.when(s_last == NSTRIP - 1)
    def _():
        pltpu.make_async_copy(
            tbuf.at[last_cur, pl.ds(0, 32)],
            tpad_hbm.at[pl.ds(0, 32)],
            last_sem,
        ).wait()


@functools.partial(
    pl.kernel,
    mesh=_mesh,
    out_type=jax.ShapeDtypeStruct((D, B), jnp.float32),
    scratch_types=[
        pltpu.VMEM((BPW,), jnp.int32),
        pltpu.VMEM((BPW, TP), jnp.float32),
        pltpu.VMEM((D, BPW), jnp.float32),
        pltpu.SemaphoreType.DMA,
    ],
    compiler_params=_params,
)
def _gather_kernel(idx_hbm, tpad_hbm, out_hbm, idx_v, rows_v, outb_v, sem):
    wid = lax.axis_index("s") * NC + lax.axis_index("c")
    base = wid * BPW
    pltpu.sync_copy(idx_hbm.at[pl.ds(base, BPW)], idx_v)
    copies = [
        pltpu.async_copy(
            tpad_hbm.at[idx_v.at[pl.ds(j * CH, CH)]],
            rows_v.at[pl.ds(j * CH, CH)],
            sem,
        )
        for j in range(NCH)
    ]
    for cp in copies:
        cp.wait()
    lane = lax.iota(jnp.int32, L)

    @plsc.parallel_loop(0, D, unroll=4)
    def _transpose(c):
        col_idx = jnp.full((L,), c, jnp.int32)
        for bg in range(BPW // L):
            row_idx = lane + (bg * L)
            v = plsc.load_gather(rows_v, [row_idx, col_idx])
            outb_v[c, pl.ds(bg * L, L)] = v

    pltpu.sync_copy(outb_v, out_hbm.at[:, pl.ds(base, BPW)])


def kernel(speaker_ids, table):
    tpad = _relayout_kernel(table.T)
    out_t = _gather_kernel(speaker_ids.astype(jnp.int32), tpad)
    return out_t.T


# speedup vs baseline: 1.1947x; 1.1947x over previous
import functools

import jax
import jax.numpy as jnp
from jax import lax
from jax.experimental import pallas as pl
from jax.experimental.pallas import tpu as pltpu
from jax.experimental.pallas import tpu_sc as plsc

D = 64
B = 16384
NC = 2
NS = 16
NW = NC * NS
BPW = B // NW
CH = 64
NCH = BPW // CH
L = 16

_mesh = plsc.VectorSubcoreMesh(core_axis_name="c", subcore_axis_name="s")
_params = pltpu.CompilerParams(
    use_tc_tiling_on_sc=True,
    needs_layout_passes=False,
    disable_bounds_checks=True,
)


@functools.partial(
    pl.kernel,
    mesh=_mesh,
    out_type=jax.ShapeDtypeStruct((D, B), jnp.float32),
    scratch_types=[
        pltpu.VMEM((BPW,), jnp.int32),
        pltpu.VMEM((CH, 8, D), jnp.float32),
        pltpu.VMEM((D, BPW), jnp.float32),
        pltpu.SemaphoreType.DMA,
        pltpu.SemaphoreType.DMA,
    ],
    compiler_params=_params,
)
def _gather_kernel(
    idx_hbm, table_hbm, out_hbm, idx_v, tiles_v, outb_v, sem, isem
):
    wid = lax.axis_index("s") * NC + lax.axis_index("c")
    base = wid * BPW
    pltpu.async_copy(idx_hbm.at[pl.ds(base, BPW)], idx_v, isem).wait()
    lane = lax.iota(jnp.int32, L)

    for j in range(NCH):
        def issue(g, _):
            iv = idx_v[pl.ds(j * CH + g * L, L)]
            for l in range(L):
                r = iv[l]
                rt = pl.multiple_of((r // 8) * 8, 8)
                pltpu.async_copy(
                    table_hbm.at[pl.ds(rt, 8), :],
                    tiles_v.at[g * L + l],
                    sem,
                )
            return _

        lax.fori_loop(0, CH // L, issue, None)
        for _ in range(CH):
            pltpu.make_async_copy(
                table_hbm.at[pl.ds(0, 8), :], tiles_v.at[0], sem
            ).wait()

        def extract(g, _):
            iv = idx_v[pl.ds(j * CH + g * L, L)]
            for l in range(L):
                b = j * CH + g * L + l
                ro = lax.rem(iv[l], 8)
                bcol = jnp.full((L,), b, jnp.int32)
                for gg in range(D // L):
                    v = tiles_v[g * L + l, ro, pl.ds(gg * L, L)]
                    plsc.store_scatter(outb_v, [lane + gg * L, bcol], v)
            return _

        lax.fori_loop(0, CH // L, extract, None)

    pltpu.sync_copy(outb_v, out_hbm.at[:, pl.ds(base, BPW)])


def kernel(speaker_ids, table):
    out_t = _gather_kernel(speaker_ids.astype(jnp.int32), table)
    return out_t.T


# speedup vs baseline: 1.2866x; 1.0769x over previous
import functools

import jax
import jax.numpy as jnp
from jax import lax
from jax.experimental import pallas as pl
from jax.experimental.pallas import tpu as pltpu
from jax.experimental.pallas import tpu_sc as plsc

D = 64
B = 16384
NC = 2
NS = 16
NW = NC * NS
BPW = B // NW
CH = 32
NCH = BPW // CH
L = 16

_mesh = plsc.VectorSubcoreMesh(core_axis_name="c", subcore_axis_name="s")
_params = pltpu.CompilerParams(
    use_tc_tiling_on_sc=True,
    needs_layout_passes=False,
    disable_bounds_checks=True,
)


@functools.partial(
    pl.kernel,
    mesh=_mesh,
    out_type=jax.ShapeDtypeStruct((D, B), jnp.float32),
    scratch_types=[
        pltpu.VMEM((BPW,), jnp.int32),
        pltpu.VMEM((2, CH, 8, D), jnp.float32),
        pltpu.VMEM((D, BPW), jnp.float32),
        pltpu.SemaphoreType.DMA,
        pltpu.SemaphoreType.DMA,
        pltpu.SemaphoreType.DMA,
    ],
    compiler_params=_params,
)
def _gather_kernel(
    idx_hbm, table_hbm, out_hbm, idx_v, tiles_v, outb_v, sem0, sem1, isem
):
    wid = lax.axis_index("s") * NC + lax.axis_index("c")
    base = wid * BPW
    pltpu.async_copy(idx_hbm.at[pl.ds(base, BPW)], idx_v, isem).wait()
    lane = lax.iota(jnp.int32, L)
    sems = (sem0, sem1)
    CHUNK_BYTES = CH * 8 * D * 4

    def issue(j):
        sem = sems[j % 2]
        buf = j % 2

        def issue_g(g, _):
            iv = idx_v[pl.ds(j * CH + g * L, L)]
            for l in range(L):
                r = iv[l]
                rt = pl.multiple_of((r // 8) * 8, 8)
                pltpu.async_copy(
                    table_hbm.at[pl.ds(rt, 8), :],
                    tiles_v.at[buf, g * L + l],
                    sem,
                )
            return _

        lax.fori_loop(0, CH // L, issue_g, None)

    issue(0)
    for j in range(NCH):
        if j + 1 < NCH:
            issue(j + 1)
        for _ in range(CH):
            pltpu.make_async_copy(
                table_hbm.at[pl.ds(0, 8), :], tiles_v.at[0, 0], sems[j % 2]
            ).wait()

        def extract(g, _):
            iv = idx_v[pl.ds(j * CH + g * L, L)]
            for l in range(L):
                b = j * CH + g * L + l
                ro = lax.rem(iv[l], 8)
                bcol = jnp.full((L,), b, jnp.int32)
                for gg in range(D // L):
                    v = tiles_v[j % 2, g * L + l, ro, pl.ds(gg * L, L)]
                    plsc.store_scatter(outb_v, [lane + gg * L, bcol], v)
            return _

        lax.fori_loop(0, CH // L, extract, None)

    pltpu.sync_copy(outb_v, out_hbm.at[:, pl.ds(base, BPW)])


def kernel(speaker_ids, table):
    out_t = _gather_kernel(speaker_ids.astype(jnp.int32), table)
    return out_t.T


# speedup vs baseline: 1.4032x; 1.0906x over previous
import functools

import jax
import jax.numpy as jnp
from jax import lax
from jax.experimental import pallas as pl
from jax.experimental.pallas import tpu as pltpu
from jax.experimental.pallas import tpu_sc as plsc

D = 64
TP = 128
B = 16384
NC = 2
NS = 16
NW = NC * NS
BPW = B // NW
CH = 128
NCH = BPW // CH
L = 16

_mesh = plsc.VectorSubcoreMesh(core_axis_name="c", subcore_axis_name="s")


@functools.partial(
    pl.kernel,
    mesh=_mesh,
    out_type=jax.ShapeDtypeStruct((D, B), jnp.float32),
    scratch_types=[
        pltpu.VMEM((BPW,), jnp.int32),
        pltpu.VMEM((BPW, TP), jnp.float32),
        pltpu.VMEM((D, BPW), jnp.float32),
        pltpu.SemaphoreType.DMA,
    ],
    compiler_params=pltpu.CompilerParams(
        use_tc_tiling_on_sc=True, needs_layout_passes=False
    ),
)
def _gather_kernel(idx_hbm, table_hbm, out_hbm, idx_v, rows_v, outb_v, sem):
    wid = lax.axis_index("s") * NC + lax.axis_index("c")
    base = wid * BPW
    pltpu.sync_copy(idx_hbm.at[pl.ds(base, BPW)], idx_v)
    copies = [
        pltpu.async_copy(
            table_hbm.at[idx_v.at[pl.ds(j * CH, CH)]],
            rows_v.at[pl.ds(j * CH, CH)],
            sem,
        )
        for j in range(NCH)
    ]
    for cp in copies:
        cp.wait()
    lane = lax.iota(jnp.int32, L)

    @plsc.parallel_loop(0, D, unroll=8)
    def _transpose(c):
        col_idx = jnp.full((L,), c, jnp.int32)
        for bg in range(BPW // L):
            row_idx = lane + (bg * L)
            v = plsc.load_gather(rows_v, [row_idx, col_idx])
            outb_v[c, pl.ds(bg * L, L)] = v
    pltpu.sync_copy(outb_v, out_hbm.at[:, pl.ds(base, BPW)])


def kernel(speaker_ids, table):
    tpad = jnp.pad(table, ((0, 0), (0, TP - D)))
    out_t = _gather_kernel(speaker_ids.astype(jnp.int32), tpad)
    return out_t.T


# speedup vs baseline: 1.4163x; 1.0094x over previous
import functools

import jax
import jax.numpy as jnp
from jax import lax
from jax.experimental import pallas as pl
from jax.experimental.pallas import tpu as pltpu
from jax.experimental.pallas import tpu_sc as plsc

D = 64
TP = 128
B = 16384
NC = 2
NS = 16
NW = NC * NS
BPW = B // NW
CH = 128
NCH = BPW // CH
L = 16

_mesh = plsc.VectorSubcoreMesh(core_axis_name="c", subcore_axis_name="s")


@functools.partial(
    pl.kernel,
    mesh=_mesh,
    out_type=jax.ShapeDtypeStruct((D, B), jnp.float32),
    scratch_types=[
        pltpu.VMEM((BPW,), jnp.int32),
        pltpu.VMEM((BPW, TP), jnp.float32),
        pltpu.VMEM((D, BPW), jnp.float32),
        pltpu.SemaphoreType.DMA,
    ],
    compiler_params=pltpu.CompilerParams(
        use_tc_tiling_on_sc=True, needs_layout_passes=False
    ),
)
def _gather_kernel(idx_hbm, table_hbm, out_hbm, idx_v, rows_v, outb_v, sem):
    wid = lax.axis_index("s") * NC + lax.axis_index("c")
    base = wid * BPW
    pltpu.sync_copy(idx_hbm.at[pl.ds(base, BPW)], idx_v)
    copies = [
        pltpu.async_copy(
            table_hbm.at[idx_v.at[pl.ds(j * CH, CH)]],
            rows_v.at[pl.ds(j * CH, CH)],
            sem,
        )
        for j in range(NCH)
    ]
    for cp in copies:
        cp.wait()
    lane = lax.iota(jnp.int32, L)

    @plsc.parallel_loop(0, D, unroll=4)
    def _transpose(c):
        col_idx = jnp.full((L,), c, jnp.int32)
        for bg in range(BPW // L):
            row_idx = lane + (bg * L)
            v = plsc.load_gather(rows_v, [row_idx, col_idx])
            outb_v[c, pl.ds(bg * L, L)] = v
    pltpu.sync_copy(outb_v, out_hbm.at[:, pl.ds(base, BPW)])


def kernel(speaker_ids, table):
    tpad = jnp.pad(table, ((0, 0), (0, TP - D)))
    out_t = _gather_kernel(speaker_ids.astype(jnp.int32), tpad)
    return out_t.T


# speedup vs baseline: 1.4231x; 1.0048x over previous
import functools

import jax
import jax.numpy as jnp
from jax import lax
from jax.experimental import pallas as pl
from jax.experimental.pallas import tpu as pltpu
from jax.experimental.pallas import tpu_sc as plsc

D = 64
TP = 128
B = 16384
NC = 2
NS = 16
NW = NC * NS
BPW = B // NW
CH = 128
NCH = BPW // CH
L = 16

_mesh = plsc.VectorSubcoreMesh(core_axis_name="c", subcore_axis_name="s")


@functools.partial(
    pl.kernel,
    mesh=_mesh,
    out_type=jax.ShapeDtypeStruct((D, B), jnp.float32),
    scratch_types=[
        pltpu.VMEM((BPW,), jnp.int32),
        pltpu.VMEM((BPW, TP), jnp.float32),
        pltpu.VMEM((D, BPW), jnp.float32),
        pltpu.SemaphoreType.DMA,
        pltpu.SemaphoreType.DMA,
        pltpu.SemaphoreType.DMA,
        pltpu.SemaphoreType.DMA,
    ],
    compiler_params=pltpu.CompilerParams(
        use_tc_tiling_on_sc=True, needs_layout_passes=False
    ),
)
def _gather_kernel(
    idx_hbm, table_hbm, out_hbm, idx_v, rows_v, outb_v, s0, s1, s2, s3
):
    wid = lax.axis_index("s") * NC + lax.axis_index("c")
    base = wid * BPW
    pltpu.sync_copy(idx_hbm.at[pl.ds(base, BPW)], idx_v)
    sems = (s0, s1, s2, s3)
    copies = [
        pltpu.async_copy(
            table_hbm.at[idx_v.at[pl.ds(j * CH, CH)]],
            rows_v.at[pl.ds(j * CH, CH)],
            sems[j],
        )
        for j in range(NCH)
    ]
    lane = lax.iota(jnp.int32, L)
    for j in range(NCH):
        copies[j].wait()

        @plsc.parallel_loop(0, D, unroll=4)
        def _transpose(c):
            col_idx = jnp.full((L,), c, jnp.int32)
            for bg in range(CH // L):
                row_idx = lane + (j * CH + bg * L)
                v = plsc.load_gather(rows_v, [row_idx, col_idx])
                outb_v[c, pl.ds(j * CH + bg * L, L)] = v

    pltpu.sync_copy(outb_v, out_hbm.at[:, pl.ds(base, BPW)])


def kernel(speaker_ids, table):
    tpad = jnp.pad(table, ((0, 0), (0, TP - D)))
    out_t = _gather_kernel(speaker_ids.astype(jnp.int32), tpad)
    return out_t.T
